# baseline (speedup 1.0000x reference)
import jax
import jax.numpy as jnp
from jax.experimental import pallas as pl
from jax.experimental.pallas import tpu as pltpu


_COLS_PER_BLOCK = 256


def _score_block(logits_ref, labels_ref, out_ref):
    x = logits_ref[...]
    lab = labels_ref[...]
    row = jax.lax.broadcasted_iota(jnp.int32, x.shape, 0)
    xl = jnp.sum(jnp.where(row == lab, x, 0.0), axis=0, keepdims=True)
    m = jnp.max(x, axis=0, keepdims=True)
    e = jnp.exp(x - m)
    z = jnp.sum(e, axis=0, keepdims=True)
    mask = (x > xl) | ((x == xl) & (row <= lab))
    num = jnp.sum(jnp.where(mask, e, 0.0), axis=0, keepdims=True)
    out_ref[...] = num / z


@jax.jit
def kernel(logits, labels):
    n, c = logits.shape
    xt = logits.T
    lab2d = labels.astype(jnp.int32).reshape(1, n)
    bn = _COLS_PER_BLOCK
    out = pl.pallas_call(
        _score_block,
        grid=(n // bn,),
        in_specs=[
            pl.BlockSpec((c, bn), lambda j: (0, j)),
            pl.BlockSpec((1, bn), lambda j: (0, j)),
        ],
        out_specs=pl.BlockSpec((1, bn), lambda j: (0, j)),
        out_shape=jax.ShapeDtypeStruct((1, n), jnp.float32),
        compiler_params=pltpu.CompilerParams(
            dimension_semantics=("parallel",),
        ),
    )(xt, lab2d)
    return out.reshape(n)


# speedup vs baseline: 1.2342x; 1.2342x over previous
import jax
import jax.numpy as jnp
from jax.experimental import pallas as pl
from jax.experimental.pallas import tpu as pltpu


_COLS_PER_BLOCK = 1024


def _score_block(logits_ref, labels_ref, out_ref):
    x = logits_ref[...]
    lab = labels_ref[...]
    row = jax.lax.broadcasted_iota(jnp.int32, x.shape, 0)
    xl = jnp.sum(jnp.where(row == lab, x, 0.0), axis=0, keepdims=True)
    m = jnp.max(x, axis=0, keepdims=True)
    e = jnp.exp(x - m)
    z = jnp.sum(e, axis=0, keepdims=True)
    mask = (x > xl) | ((x == xl) & (row <= lab))
    num = jnp.sum(jnp.where(mask, e, 0.0), axis=0, keepdims=True)
    out_ref[...] = num / z


@jax.jit
def kernel(logits, labels):
    n, c = logits.shape
    xt = logits.T
    lab2d = labels.astype(jnp.int32).reshape(1, n)
    bn = _COLS_PER_BLOCK
    out = pl.pallas_call(
        _score_block,
        grid=(n // bn,),
        in_specs=[
            pl.BlockSpec((c, bn), lambda j: (0, j)),
            pl.BlockSpec((1, bn), lambda j: (0, j)),
        ],
        out_specs=pl.BlockSpec((1, bn), lambda j: (0, j)),
        out_shape=jax.ShapeDtypeStruct((1, n), jnp.float32),
        compiler_params=pltpu.CompilerParams(
            dimension_semantics=("parallel",),
        ),
    )(xt, lab2d)
    return out.reshape(n)
